# baseline (speedup 1.0000x reference)
import jax
import jax.numpy as jnp
from jax import lax
from jax.experimental import pallas as pl
from jax.experimental.pallas import tpu as pltpu
from jax.experimental.pallas import tpu_sc as plsc

_T = 26
_V = 100000
_D = 64
_B = 4096
_N = _B * _T
_NC = 2
_NS = 16
_NW = _NC * _NS
_PER_W = _N // _NW
_CHUNK = 128
_NCH = _PER_W // _CHUNK
_VREGS = _PER_W // 16


def _body(idx_hbm, tab_hbm, out_hbm, idx_v, flat_v, rows_v, gsem, wsem):
    wid = lax.axis_index("s") * _NC + lax.axis_index("c")
    base = wid * _PER_W
    pltpu.sync_copy(idx_hbm.at[pl.ds(base, _PER_W)], idx_v)

    def vr(i, t0):
        tvec = t0 + lax.iota(jnp.int32, 16)
        tvec = jnp.where(tvec >= _T, tvec - _T, tvec)
        vals = idx_v[pl.ds(i * 16, 16)] + tvec * _V
        flat_v[lax.shift_right_logical(i, 3), pl.ds((i & 7) * 16, 16)] = vals
        t0 = t0 + 16
        return jnp.where(t0 >= _T, t0 - _T, t0)

    lax.fori_loop(0, _VREGS, vr, jnp.int32(0))

    pltpu.async_copy(tab_hbm.at[flat_v.at[0]], rows_v.at[0], gsem)

    def ch(j, _):
        slot = j & 1
        pltpu.make_async_copy(
            tab_hbm.at[flat_v.at[j]], rows_v.at[slot], gsem).wait()

        @pl.when(j >= 1)
        def _():
            pltpu.make_async_copy(
                rows_v.at[1 - slot],
                out_hbm.at[pl.ds(base + (j - 1) * _CHUNK, _CHUNK)],
                wsem).wait()

        @pl.when(j + 1 < _NCH)
        def _():
            pltpu.async_copy(
                tab_hbm.at[flat_v.at[j + 1]], rows_v.at[1 - slot], gsem)

        pltpu.async_copy(
            rows_v.at[slot],
            out_hbm.at[pl.ds(base + j * _CHUNK, _CHUNK)], wsem)
        return 0

    lax.fori_loop(0, _NCH, ch, 0)

    pltpu.make_async_copy(
        rows_v.at[(_NCH - 1) & 1],
        out_hbm.at[pl.ds(base + (_NCH - 1) * _CHUNK, _CHUNK)], wsem).wait()


_mesh = plsc.VectorSubcoreMesh(core_axis_name="c", subcore_axis_name="s")

_gather = pl.kernel(
    _body,
    out_type=jax.ShapeDtypeStruct((_N, _D), jnp.float32),
    mesh=_mesh,
    scratch_types=[
        pltpu.VMEM((_PER_W,), jnp.int32),
        pltpu.VMEM((_NCH, _CHUNK), jnp.int32),
        pltpu.VMEM((2, _CHUNK, _D), jnp.float32),
        pltpu.SemaphoreType.DMA,
        pltpu.SemaphoreType.DMA,
    ],
    compiler_params=pltpu.CompilerParams(use_tc_tiling_on_sc=False),
)


@jax.jit
def kernel(inputs, tables):
    idx = inputs.astype(jnp.int32).reshape(_N)
    tab = tables.reshape(_T * _V, _D)
    out = _gather(idx, tab)
    return out.reshape(_B, _T, _D)


# speedup vs baseline: 1.0085x; 1.0085x over previous
import jax
import jax.numpy as jnp
from jax import lax
from jax.experimental import pallas as pl
from jax.experimental.pallas import tpu as pltpu
from jax.experimental.pallas import tpu_sc as plsc

_T = 26
_V = 100000
_D = 64
_B = 4096
_N = _B * _T
_NC = 2
_NS = 16
_NW = _NC * _NS
_PER_W = _N // _NW
_CHUNK = 128
_NCH = _PER_W // _CHUNK
_VREGS = _PER_W // 16
_K = 8
_G = 6


def _body(idx_hbm, tab_hbm, out_hbm, idx_v, flat_v, rows_v, gsem, wsem):
    wid = lax.axis_index("s") * _NC + lax.axis_index("c")
    base = wid * _PER_W
    pltpu.sync_copy(idx_hbm.at[pl.ds(base, _PER_W)], idx_v)

    def vr(i, t0):
        tvec = t0 + lax.iota(jnp.int32, 16)
        tvec = jnp.where(tvec >= _T, tvec - _T, tvec)
        vals = idx_v[pl.ds(i * 16, 16)] + tvec * _V
        flat_v[lax.shift_right_logical(i, 3), pl.ds((i & 7) * 16, 16)] = vals
        t0 = t0 + 16
        return jnp.where(t0 >= _T, t0 - _T, t0)

    lax.fori_loop(0, _VREGS, vr, jnp.int32(0))

    for b in range(_G):
        pltpu.async_copy(tab_hbm.at[flat_v.at[b]], rows_v.at[b], gsem)

    def ch(j, _):
        s = j & (_K - 1)
        pltpu.make_async_copy(
            tab_hbm.at[flat_v.at[j]], rows_v.at[s], gsem).wait()
        pltpu.async_copy(
            rows_v.at[s],
            out_hbm.at[pl.ds(base + j * _CHUNK, _CHUNK)], wsem)

        nj = j + _G

        @pl.when(nj < _NCH)
        def _():
            @pl.when(nj >= _K)
            def _():
                pltpu.make_async_copy(
                    rows_v.at[nj & (_K - 1)],
                    out_hbm.at[pl.ds(base + (nj - _K) * _CHUNK, _CHUNK)],
                    wsem).wait()

            pltpu.async_copy(
                tab_hbm.at[flat_v.at[nj]], rows_v.at[nj & (_K - 1)], gsem)

        return 0

    lax.fori_loop(0, _NCH, ch, 0)

    def dr(j, _):
        pltpu.make_async_copy(
            rows_v.at[j & (_K - 1)],
            out_hbm.at[pl.ds(base + j * _CHUNK, _CHUNK)], wsem).wait()
        return 0

    lax.fori_loop(_NCH - _K, _NCH, dr, 0)


_mesh = plsc.VectorSubcoreMesh(core_axis_name="c", subcore_axis_name="s")

_gather = pl.kernel(
    _body,
    out_type=jax.ShapeDtypeStruct((_N, _D), jnp.float32),
    mesh=_mesh,
    scratch_types=[
        pltpu.VMEM((_PER_W,), jnp.int32),
        pltpu.VMEM((_NCH, _CHUNK), jnp.int32),
        pltpu.VMEM((_K, _CHUNK, _D), jnp.float32),
        pltpu.SemaphoreType.DMA,
        pltpu.SemaphoreType.DMA,
    ],
    compiler_params=pltpu.CompilerParams(use_tc_tiling_on_sc=False),
)


@jax.jit
def kernel(inputs, tables):
    idx = inputs.astype(jnp.int32).reshape(_N)
    tab = tables.reshape(_T * _V, _D)
    out = _gather(idx, tab)
    return out.reshape(_B, _T, _D)
